# baseline (speedup 1.0000x reference)
import functools

import jax
import jax.numpy as jnp
from jax import lax
from jax.experimental import pallas as pl
from jax.experimental.pallas import tpu as pltpu
from jax.experimental.pallas import tpu_sc as plsc

_B, _S, _H = 4, 2048, 1024
_NUM_SELECTED = 256
_NW = 32
_WORKERS_PER_BATCH = _NW // _B
_ROWS_PER_WORKER = _S // _WORKERS_PER_BATCH
_CH = 32
_NCHUNK = _ROWS_PER_WORKER // _CH


@functools.partial(
    pl.kernel,
    mesh=plsc.VectorSubcoreMesh(core_axis_name="c", subcore_axis_name="s"),
    out_type=[
        jax.ShapeDtypeStruct((_B, _WORKERS_PER_BATCH, _H), jnp.float32),
        jax.ShapeDtypeStruct((_B, _H), jnp.float32),
    ],
    scratch_types=[
        pltpu.VMEM((2, _CH, _H), jnp.float32),
        pltpu.VMEM((_H,), jnp.float32),
        pltpu.SemaphoreType.DMA,
        pltpu.SemaphoreType.DMA,
    ],
)
def _sc_sum(h_hbm, part_out, last_out, buf, acc, sem0, sem1):
    c = lax.axis_index("c")
    s = lax.axis_index("s")
    wid = c * 16 + s
    b = wid // _WORKERS_PER_BATCH
    widx = wid % _WORKERS_PER_BATCH
    r0 = widx * _ROWS_PER_WORKER
    sems = [sem0, sem1]

    for i in range(_H // 16):
        acc[pl.ds(i * 16, 16)] = jnp.zeros((16,), jnp.float32)

    def src(chunk):
        return h_hbm.at[b, pl.ds(r0 + chunk * _CH, _CH)]

    def accumulate(slot):
        def body(r, carry):
            for col in range(_H // 16):
                sl = pl.ds(col * 16, 16)
                acc[sl] = acc[sl] + buf[slot, r, sl]
            return carry
        lax.fori_loop(0, _CH, body, 0)

    handles = {0: pltpu.async_copy(src(0), buf.at[0], sems[0])}
    for chunk in range(_NCHUNK):
        slot = chunk % 2
        if chunk + 1 < _NCHUNK:
            handles[chunk + 1] = pltpu.async_copy(
                src(chunk + 1), buf.at[1 - slot], sems[1 - slot])
        handles[chunk].wait()
        accumulate(slot)

    pltpu.sync_copy(acc, part_out.at[b, widx])

    @pl.when(widx == _WORKERS_PER_BATCH - 1)
    def _emit_last():
        last_slot = (_NCHUNK - 1) % 2
        pltpu.sync_copy(buf.at[last_slot, _CH - 1], last_out.at[b])


def _finalize_kernel(p_ref, last_ref, wm_ref, bm_ref, wd_ref, bd_ref,
                     mask_out_ref, delta_out_ref, *, seq_len, num_selected):
    sums = jnp.sum(p_ref[...], axis=1)
    mean_h = sums * (1.0 / seq_len)
    mask_out_ref[...] = (
        jnp.dot(last_ref[...], wm_ref[...],
                preferred_element_type=jnp.float32) + bm_ref[...]
    )
    delta_rows = (
        jnp.dot(mean_h, wd_ref[...],
                preferred_element_type=jnp.float32) + bd_ref[...]
    )
    delta_out_ref[...] = jnp.broadcast_to(
        delta_rows[:, None, :],
        (delta_rows.shape[0], num_selected, delta_rows.shape[1]),
    )


@jax.jit
def _edit_head(hidden_states, W_mask, b_mask, W_delta, b_delta):
    B, S, H = hidden_states.shape
    M = W_mask.shape[1]
    D = W_delta.shape[1]

    parts, lasts = _sc_sum(hidden_states)

    mask_flat, edit_delta = pl.pallas_call(
        functools.partial(_finalize_kernel, seq_len=S,
                          num_selected=_NUM_SELECTED),
        grid=(1,),
        in_specs=[
            pl.BlockSpec((B, _WORKERS_PER_BATCH, H), lambda i: (0, 0, 0)),
            pl.BlockSpec((B, H), lambda i: (0, 0)),
            pl.BlockSpec((H, M), lambda i: (0, 0)),
            pl.BlockSpec((M,), lambda i: (0,)),
            pl.BlockSpec((H, D), lambda i: (0, 0)),
            pl.BlockSpec((D,), lambda i: (0,)),
        ],
        out_specs=[
            pl.BlockSpec((B, M), lambda i: (0, 0)),
            pl.BlockSpec((B, _NUM_SELECTED, D), lambda i: (0, 0, 0)),
        ],
        out_shape=[
            jax.ShapeDtypeStruct((B, M), jnp.float32),
            jax.ShapeDtypeStruct((B, _NUM_SELECTED, D), jnp.float32),
        ],
    )(parts, lasts, W_mask, b_mask, W_delta, b_delta)

    mask_size = int(round(M ** 0.5))
    sparse_mask = mask_flat.reshape(B, mask_size, mask_size)
    return sparse_mask, edit_delta


def kernel(hidden_states, W_mask, b_mask, W_delta, b_delta, num_selected):
    del num_selected
    return _edit_head(hidden_states, W_mask, b_mask, W_delta, b_delta)


# speedup vs baseline: 1.1955x; 1.1955x over previous
import functools

import jax
import jax.numpy as jnp
from jax import lax
from jax.experimental import pallas as pl
from jax.experimental.pallas import tpu as pltpu
from jax.experimental.pallas import tpu_sc as plsc

_B, _S, _H = 4, 2048, 1024
_NUM_SELECTED = 256
_NW = 32
_WORKERS_PER_BATCH = _NW // _B
_ROWS_PER_WORKER = _S // _WORKERS_PER_BATCH
_CH = 32
_NCHUNK = _ROWS_PER_WORKER // _CH


@functools.partial(
    pl.kernel,
    mesh=plsc.VectorSubcoreMesh(core_axis_name="c", subcore_axis_name="s"),
    out_type=[
        jax.ShapeDtypeStruct((_B, _WORKERS_PER_BATCH, _H), jnp.float32),
        jax.ShapeDtypeStruct((_B, _H), jnp.float32),
    ],
    scratch_types=[
        pltpu.VMEM((2, _CH, _H), jnp.float32),
        pltpu.VMEM((_H,), jnp.float32),
        pltpu.SemaphoreType.DMA,
        pltpu.SemaphoreType.DMA,
    ],
)
def _sc_sum(h_hbm, part_out, last_out, buf, acc, sem0, sem1):
    c = lax.axis_index("c")
    s = lax.axis_index("s")
    wid = c * 16 + s
    b = wid // _WORKERS_PER_BATCH
    widx = wid % _WORKERS_PER_BATCH
    r0 = widx * _ROWS_PER_WORKER
    sems = [sem0, sem1]

    for i in range(_H // 16):
        acc[pl.ds(i * 16, 16)] = jnp.zeros((16,), jnp.float32)

    def src(chunk):
        return h_hbm.at[b, pl.ds(r0 + chunk * _CH, _CH)]

    def accumulate(slot):
        def body(r, carry):
            for col in range(_H // 16):
                sl = pl.ds(col * 16, 16)
                plsc.addupdate(acc.at[sl], buf[slot, r, sl])
            return carry
        lax.fori_loop(0, _CH, body, 0)

    handles = {0: pltpu.async_copy(src(0), buf.at[0], sems[0])}
    for chunk in range(_NCHUNK):
        slot = chunk % 2
        if chunk + 1 < _NCHUNK:
            handles[chunk + 1] = pltpu.async_copy(
                src(chunk + 1), buf.at[1 - slot], sems[1 - slot])
        handles[chunk].wait()
        accumulate(slot)

    pltpu.sync_copy(acc, part_out.at[b, widx])

    @pl.when(widx == _WORKERS_PER_BATCH - 1)
    def _emit_last():
        last_slot = (_NCHUNK - 1) % 2
        pltpu.sync_copy(buf.at[last_slot, _CH - 1], last_out.at[b])


def _finalize_kernel(p_ref, last_ref, wm_ref, bm_ref, wd_ref, bd_ref,
                     mask_out_ref, delta_out_ref, *, seq_len, num_selected):
    sums = jnp.sum(p_ref[...], axis=1)
    mean_h = sums * (1.0 / seq_len)
    mask_out_ref[...] = (
        jnp.dot(last_ref[...], wm_ref[...],
                preferred_element_type=jnp.float32) + bm_ref[...]
    )
    delta_rows = (
        jnp.dot(mean_h, wd_ref[...],
                preferred_element_type=jnp.float32) + bd_ref[...]
    )
    delta_out_ref[...] = jnp.broadcast_to(
        delta_rows[:, None, :],
        (delta_rows.shape[0], num_selected, delta_rows.shape[1]),
    )


@jax.jit
def _edit_head(hidden_states, W_mask, b_mask, W_delta, b_delta):
    B, S, H = hidden_states.shape
    M = W_mask.shape[1]
    D = W_delta.shape[1]

    parts, lasts = _sc_sum(hidden_states)

    mask_flat, edit_delta = pl.pallas_call(
        functools.partial(_finalize_kernel, seq_len=S,
                          num_selected=_NUM_SELECTED),
        grid=(1,),
        in_specs=[
            pl.BlockSpec((B, _WORKERS_PER_BATCH, H), lambda i: (0, 0, 0)),
            pl.BlockSpec((B, H), lambda i: (0, 0)),
            pl.BlockSpec((H, M), lambda i: (0, 0)),
            pl.BlockSpec((M,), lambda i: (0,)),
            pl.BlockSpec((H, D), lambda i: (0, 0)),
            pl.BlockSpec((D,), lambda i: (0,)),
        ],
        out_specs=[
            pl.BlockSpec((B, M), lambda i: (0, 0)),
            pl.BlockSpec((B, _NUM_SELECTED, D), lambda i: (0, 0, 0)),
        ],
        out_shape=[
            jax.ShapeDtypeStruct((B, M), jnp.float32),
            jax.ShapeDtypeStruct((B, _NUM_SELECTED, D), jnp.float32),
        ],
    )(parts, lasts, W_mask, b_mask, W_delta, b_delta)

    mask_size = int(round(M ** 0.5))
    sparse_mask = mask_flat.reshape(B, mask_size, mask_size)
    return sparse_mask, edit_delta


def kernel(hidden_states, W_mask, b_mask, W_delta, b_delta, num_selected):
    del num_selected
    return _edit_head(hidden_states, W_mask, b_mask, W_delta, b_delta)


# speedup vs baseline: 2.3253x; 1.9450x over previous
import functools

import jax
import jax.numpy as jnp
from jax import lax
from jax.experimental import pallas as pl
from jax.experimental.pallas import tpu as pltpu
from jax.experimental.pallas import tpu_sc as plsc

_B, _S, _H = 4, 2048, 1024
_NUM_SELECTED = 256
_NW = 32
_WORKERS_PER_BATCH = _NW // _B
_ROWS_PER_WORKER = _S // _WORKERS_PER_BATCH
_CH = 32
_NCHUNK = _ROWS_PER_WORKER // _CH


@functools.partial(
    pl.kernel,
    mesh=plsc.VectorSubcoreMesh(core_axis_name="c", subcore_axis_name="s"),
    out_type=[
        jax.ShapeDtypeStruct((_B, _WORKERS_PER_BATCH, _H), jnp.float32),
        jax.ShapeDtypeStruct((_B, _H), jnp.float32),
    ],
    scratch_types=[
        pltpu.VMEM((2, _CH, _H), jnp.float32),
        pltpu.VMEM((_H,), jnp.float32),
        pltpu.SemaphoreType.DMA,
        pltpu.SemaphoreType.DMA,
    ],
)
def _sc_sum(h_hbm, part_out, last_out, buf, acc, sem0, sem1):
    c = lax.axis_index("c")
    s = lax.axis_index("s")
    wid = c * 16 + s
    b = wid // _WORKERS_PER_BATCH
    widx = wid % _WORKERS_PER_BATCH
    r0 = widx * _ROWS_PER_WORKER
    sems = [sem0, sem1]

    for i in range(_H // 16):
        acc[pl.ds(i * 16, 16)] = jnp.zeros((16,), jnp.float32)

    def src(chunk):
        return h_hbm.at[b, pl.ds(r0 + chunk * _CH, _CH)]

    def accumulate(slot):
        def body(col):
            sl = pl.ds(col * 16, 16)
            vals = [buf[slot, r, sl] for r in range(_CH)]
            while len(vals) > 1:
                nxt = [vals[i] + vals[i + 1]
                       for i in range(0, len(vals) - 1, 2)]
                if len(vals) % 2:
                    nxt.append(vals[-1])
                vals = nxt
            plsc.addupdate(acc.at[sl], vals[0])
        plsc.parallel_loop(0, _H // 16, 1, unroll=2)(body)

    handles = {0: pltpu.async_copy(src(0), buf.at[0], sems[0])}
    for chunk in range(_NCHUNK):
        slot = chunk % 2
        if chunk + 1 < _NCHUNK:
            handles[chunk + 1] = pltpu.async_copy(
                src(chunk + 1), buf.at[1 - slot], sems[1 - slot])
        handles[chunk].wait()
        accumulate(slot)

    pltpu.sync_copy(acc, part_out.at[b, widx])

    @pl.when(widx == _WORKERS_PER_BATCH - 1)
    def _emit_last():
        last_slot = (_NCHUNK - 1) % 2
        pltpu.sync_copy(buf.at[last_slot, _CH - 1], last_out.at[b])


def _finalize_kernel(p_ref, last_ref, wm_ref, bm_ref, wd_ref, bd_ref,
                     mask_out_ref, delta_out_ref, *, seq_len, num_selected):
    sums = jnp.sum(p_ref[...], axis=1)
    mean_h = sums * (1.0 / seq_len)
    mask_out_ref[...] = (
        jnp.dot(last_ref[...], wm_ref[...],
                preferred_element_type=jnp.float32) + bm_ref[...]
    )
    delta_rows = (
        jnp.dot(mean_h, wd_ref[...],
                preferred_element_type=jnp.float32) + bd_ref[...]
    )
    delta_out_ref[...] = jnp.broadcast_to(
        delta_rows[:, None, :],
        (delta_rows.shape[0], num_selected, delta_rows.shape[1]),
    )


@jax.jit
def _edit_head(hidden_states, W_mask, b_mask, W_delta, b_delta):
    B, S, H = hidden_states.shape
    M = W_mask.shape[1]
    D = W_delta.shape[1]

    parts, lasts = _sc_sum(hidden_states)

    mask_flat, edit_delta = pl.pallas_call(
        functools.partial(_finalize_kernel, seq_len=S,
                          num_selected=_NUM_SELECTED),
        grid=(1,),
        in_specs=[
            pl.BlockSpec((B, _WORKERS_PER_BATCH, H), lambda i: (0, 0, 0)),
            pl.BlockSpec((B, H), lambda i: (0, 0)),
            pl.BlockSpec((H, M), lambda i: (0, 0)),
            pl.BlockSpec((M,), lambda i: (0,)),
            pl.BlockSpec((H, D), lambda i: (0, 0)),
            pl.BlockSpec((D,), lambda i: (0,)),
        ],
        out_specs=[
            pl.BlockSpec((B, M), lambda i: (0, 0)),
            pl.BlockSpec((B, _NUM_SELECTED, D), lambda i: (0, 0, 0)),
        ],
        out_shape=[
            jax.ShapeDtypeStruct((B, M), jnp.float32),
            jax.ShapeDtypeStruct((B, _NUM_SELECTED, D), jnp.float32),
        ],
    )(parts, lasts, W_mask, b_mask, W_delta, b_delta)

    mask_size = int(round(M ** 0.5))
    sparse_mask = mask_flat.reshape(B, mask_size, mask_size)
    return sparse_mask, edit_delta


def kernel(hidden_states, W_mask, b_mask, W_delta, b_delta, num_selected):
    del num_selected
    return _edit_head(hidden_states, W_mask, b_mask, W_delta, b_delta)


# speedup vs baseline: 2.3696x; 1.0191x over previous
import functools

import jax
import jax.numpy as jnp
from jax import lax
from jax.experimental import pallas as pl
from jax.experimental.pallas import tpu as pltpu
from jax.experimental.pallas import tpu_sc as plsc

_B, _S, _H = 4, 2048, 1024
_NUM_SELECTED = 256
_NW = 32
_WORKERS_PER_BATCH = _NW // _B
_ROWS_PER_WORKER = _S // _WORKERS_PER_BATCH
_CH = 32
_NCHUNK = _ROWS_PER_WORKER // _CH


@functools.partial(
    pl.kernel,
    mesh=plsc.VectorSubcoreMesh(core_axis_name="c", subcore_axis_name="s"),
    out_type=[
        jax.ShapeDtypeStruct((_B, _WORKERS_PER_BATCH, _H), jnp.float32),
        jax.ShapeDtypeStruct((_B, _H), jnp.float32),
    ],
    scratch_types=[
        pltpu.VMEM((3, _CH, _H), jnp.float32),
        pltpu.VMEM((_H,), jnp.float32),
        pltpu.SemaphoreType.DMA,
        pltpu.SemaphoreType.DMA,
        pltpu.SemaphoreType.DMA,
    ],
)
def _sc_sum(h_hbm, part_out, last_out, buf, acc, sem0, sem1, sem2):
    c = lax.axis_index("c")
    s = lax.axis_index("s")
    wid = c * 16 + s
    b = wid // _WORKERS_PER_BATCH
    widx = wid % _WORKERS_PER_BATCH
    r0 = widx * _ROWS_PER_WORKER
    sems = [sem0, sem1, sem2]
    nbuf = 3

    for i in range(_H // 16):
        acc[pl.ds(i * 16, 16)] = jnp.zeros((16,), jnp.float32)

    def src(chunk):
        return h_hbm.at[b, pl.ds(r0 + chunk * _CH, _CH)]

    def accumulate(slot):
        def body(col):
            sl = pl.ds(col * 16, 16)
            vals = [buf[slot, r, sl] for r in range(_CH)]
            while len(vals) > 1:
                nxt = [vals[i] + vals[i + 1]
                       for i in range(0, len(vals) - 1, 2)]
                if len(vals) % 2:
                    nxt.append(vals[-1])
                vals = nxt
            plsc.addupdate(acc.at[sl], vals[0])
        plsc.parallel_loop(0, _H // 16, 1, unroll=2)(body)

    handles = {}
    for chunk in range(min(nbuf, _NCHUNK)):
        handles[chunk] = pltpu.async_copy(
            src(chunk), buf.at[chunk % nbuf], sems[chunk % nbuf])
    for chunk in range(_NCHUNK):
        slot = chunk % nbuf
        handles[chunk].wait()
        accumulate(slot)
        nxt = chunk + nbuf
        if nxt < _NCHUNK:
            handles[nxt] = pltpu.async_copy(
                src(nxt), buf.at[slot], sems[slot])

    pltpu.sync_copy(acc, part_out.at[b, widx])

    @pl.when(widx == _WORKERS_PER_BATCH - 1)
    def _emit_last():
        last_slot = (_NCHUNK - 1) % nbuf
        pltpu.sync_copy(buf.at[last_slot, _CH - 1], last_out.at[b])


def _finalize_kernel(p_ref, last_ref, wm_ref, bm_ref, wd_ref, bd_ref,
                     mask_out_ref, delta_out_ref, *, seq_len, num_selected):
    sums = jnp.sum(p_ref[...], axis=1)
    mean_h = sums * (1.0 / seq_len)
    mask_out_ref[...] = (
        jnp.dot(last_ref[...], wm_ref[...],
                preferred_element_type=jnp.float32) + bm_ref[...]
    )
    delta_rows = (
        jnp.dot(mean_h, wd_ref[...],
                preferred_element_type=jnp.float32) + bd_ref[...]
    )
    delta_out_ref[...] = jnp.broadcast_to(
        delta_rows[:, None, :],
        (delta_rows.shape[0], num_selected, delta_rows.shape[1]),
    )


@jax.jit
def _edit_head(hidden_states, W_mask, b_mask, W_delta, b_delta):
    B, S, H = hidden_states.shape
    M = W_mask.shape[1]
    D = W_delta.shape[1]

    parts, lasts = _sc_sum(hidden_states)

    mask_flat, edit_delta = pl.pallas_call(
        functools.partial(_finalize_kernel, seq_len=S,
                          num_selected=_NUM_SELECTED),
        grid=(1,),
        in_specs=[
            pl.BlockSpec((B, _WORKERS_PER_BATCH, H), lambda i: (0, 0, 0)),
            pl.BlockSpec((B, H), lambda i: (0, 0)),
            pl.BlockSpec((H, M), lambda i: (0, 0)),
            pl.BlockSpec((M,), lambda i: (0,)),
            pl.BlockSpec((H, D), lambda i: (0, 0)),
            pl.BlockSpec((D,), lambda i: (0,)),
        ],
        out_specs=[
            pl.BlockSpec((B, M), lambda i: (0, 0)),
            pl.BlockSpec((B, _NUM_SELECTED, D), lambda i: (0, 0, 0)),
        ],
        out_shape=[
            jax.ShapeDtypeStruct((B, M), jnp.float32),
            jax.ShapeDtypeStruct((B, _NUM_SELECTED, D), jnp.float32),
        ],
    )(parts, lasts, W_mask, b_mask, W_delta, b_delta)

    mask_size = int(round(M ** 0.5))
    sparse_mask = mask_flat.reshape(B, mask_size, mask_size)
    return sparse_mask, edit_delta


def kernel(hidden_states, W_mask, b_mask, W_delta, b_delta, num_selected):
    del num_selected
    return _edit_head(hidden_states, W_mask, b_mask, W_delta, b_delta)


# speedup vs baseline: 3.0795x; 1.2996x over previous
import functools

import jax
import jax.numpy as jnp
from jax import lax
from jax.experimental import pallas as pl
from jax.experimental.pallas import tpu as pltpu
from jax.experimental.pallas import tpu_sc as plsc

_B, _S, _H = 4, 2048, 1024
_NUM_SELECTED = 256
_NW = 32
_WORKERS_PER_BATCH = _NW // _B
_SC_ROWS = 512
_TC_ROWS = _S - _SC_ROWS
_ROWS_PER_WORKER = _SC_ROWS // _WORKERS_PER_BATCH
_CH = 32
_NCHUNK = _ROWS_PER_WORKER // _CH
_NBUF = 2


@functools.partial(
    pl.kernel,
    mesh=plsc.VectorSubcoreMesh(core_axis_name="c", subcore_axis_name="s"),
    out_type=[
        jax.ShapeDtypeStruct((_B, _WORKERS_PER_BATCH, _H), jnp.float32),
    ],
    scratch_types=[
        pltpu.VMEM((_NBUF, _CH, _H), jnp.float32),
        pltpu.VMEM((_H,), jnp.float32),
        pltpu.SemaphoreType.DMA,
        pltpu.SemaphoreType.DMA,
    ],
)
def _sc_tail_sum(h_hbm, part_out, buf, acc, sem0, sem1):
    c = lax.axis_index("c")
    s = lax.axis_index("s")
    wid = c * 16 + s
    b = wid // _WORKERS_PER_BATCH
    widx = wid % _WORKERS_PER_BATCH
    r0 = _TC_ROWS + widx * _ROWS_PER_WORKER
    sems = [sem0, sem1]

    for i in range(_H // 16):
        acc[pl.ds(i * 16, 16)] = jnp.zeros((16,), jnp.float32)

    def src(chunk):
        return h_hbm.at[b, pl.ds(r0 + chunk * _CH, _CH)]

    def accumulate(slot):
        def body(col):
            sl = pl.ds(col * 16, 16)
            vals = [buf[slot, r, sl] for r in range(_CH)]
            while len(vals) > 1:
                nxt = [vals[i] + vals[i + 1]
                       for i in range(0, len(vals) - 1, 2)]
                if len(vals) % 2:
                    nxt.append(vals[-1])
                vals = nxt
            plsc.addupdate(acc.at[sl], vals[0])
        plsc.parallel_loop(0, _H // 16, 1, unroll=2)(body)

    handles = {}
    for chunk in range(min(_NBUF, _NCHUNK)):
        handles[chunk] = pltpu.async_copy(
            src(chunk), buf.at[chunk % _NBUF], sems[chunk % _NBUF])
    for chunk in range(_NCHUNK):
        slot = chunk % _NBUF
        handles[chunk].wait()
        accumulate(slot)
        nxt = chunk + _NBUF
        if nxt < _NCHUNK:
            handles[nxt] = pltpu.async_copy(
                src(nxt), buf.at[slot], sems[slot])

    pltpu.sync_copy(acc, part_out.at[b, widx])


def _tc_head_kernel(h_ref, last_ref, wm_ref, bm_ref,
                    sums_out_ref, mask_out_ref, *, n_batches):
    i = pl.program_id(0)
    h = h_ref[0]
    sums_out_ref[0] = jnp.sum(h, axis=0, keepdims=True)

    @pl.when(i == n_batches - 1)
    def _mask():
        mask_out_ref[...] = (
            jnp.dot(last_ref[...], wm_ref[...],
                    preferred_element_type=jnp.float32) + bm_ref[...]
        )


def _tc_delta_kernel(tc_sums_ref, sc_parts_ref, wd_ref, bd_ref,
                     delta_out_ref, *, seq_len, num_selected):
    total = tc_sums_ref[:, 0, :] + jnp.sum(sc_parts_ref[...], axis=1)
    mean_h = total * (1.0 / seq_len)
    delta_rows = (
        jnp.dot(mean_h, wd_ref[...],
                preferred_element_type=jnp.float32) + bd_ref[...]
    )
    delta_out_ref[...] = jnp.broadcast_to(
        delta_rows[:, None, :],
        (delta_rows.shape[0], num_selected, delta_rows.shape[1]),
    )


@jax.jit
def _edit_head(hidden_states, W_mask, b_mask, W_delta, b_delta):
    B, S, H = hidden_states.shape
    M = W_mask.shape[1]
    D = W_delta.shape[1]

    lasts = hidden_states[:, -1]

    sc_parts = _sc_tail_sum(hidden_states)[0]

    tc_sums, mask_flat = pl.pallas_call(
        functools.partial(_tc_head_kernel, n_batches=B),
        grid=(B,),
        in_specs=[
            pl.BlockSpec((1, _TC_ROWS, H), lambda i: (i, 0, 0)),
            pl.BlockSpec((B, H), lambda i: (0, 0)),
            pl.BlockSpec((H, M), lambda i: (0, 0)),
            pl.BlockSpec((M,), lambda i: (0,)),
        ],
        out_specs=[
            pl.BlockSpec((1, 1, H), lambda i: (i, 0, 0)),
            pl.BlockSpec((B, M), lambda i: (0, 0)),
        ],
        out_shape=[
            jax.ShapeDtypeStruct((B, 1, H), jnp.float32),
            jax.ShapeDtypeStruct((B, M), jnp.float32),
        ],
    )(hidden_states, lasts, W_mask, b_mask)

    edit_delta = pl.pallas_call(
        functools.partial(_tc_delta_kernel, seq_len=S,
                          num_selected=_NUM_SELECTED),
        grid=(1,),
        in_specs=[
            pl.BlockSpec((B, 1, H), lambda i: (0, 0, 0)),
            pl.BlockSpec((B, _WORKERS_PER_BATCH, H), lambda i: (0, 0, 0)),
            pl.BlockSpec((H, D), lambda i: (0, 0)),
            pl.BlockSpec((D,), lambda i: (0,)),
        ],
        out_specs=pl.BlockSpec((B, _NUM_SELECTED, D), lambda i: (0, 0, 0)),
        out_shape=jax.ShapeDtypeStruct((B, _NUM_SELECTED, D), jnp.float32),
    )(tc_sums, sc_parts, W_delta, b_delta)

    mask_size = int(round(M ** 0.5))
    sparse_mask = mask_flat.reshape(B, mask_size, mask_size)
    return sparse_mask, edit_delta


def kernel(hidden_states, W_mask, b_mask, W_delta, b_delta, num_selected):
    del num_selected
    return _edit_head(hidden_states, W_mask, b_mask, W_delta, b_delta)


# speedup vs baseline: 5.6390x; 1.8312x over previous
import functools

import jax
import jax.numpy as jnp
from jax.experimental import pallas as pl
from jax.experimental.pallas import tpu as pltpu

_NUM_SELECTED_STATIC = 256


def _edit_head_kernel(h_ref, wm_ref, bm_ref, wd_ref, bd_ref,
                      mask_out_ref, delta_out_ref, acc_ref,
                      *, seq_len, num_selected_static):
    j = pl.program_id(1)
    h = h_ref[0]
    part = jnp.sum(h, axis=0, keepdims=True)

    @pl.when(j == 0)
    def _first_half():
        acc_ref[...] = part

    @pl.when(j == 1)
    def _second_half():
        mean_h = (acc_ref[...] + part) * (1.0 / seq_len)

        last_hidden = h[-1:, :]
        mask_out_ref[0] = (
            jnp.dot(last_hidden, wm_ref[...],
                    preferred_element_type=jnp.float32) + bm_ref[...]
        )

        delta_row = (
            jnp.dot(mean_h, wd_ref[...],
                    preferred_element_type=jnp.float32) + bd_ref[...]
        )
        delta_out_ref[...] = jnp.broadcast_to(
            delta_row[:, None, :],
            (1, num_selected_static, delta_row.shape[1]),
        )


@jax.jit
def _edit_head(hidden_states, W_mask, b_mask, W_delta, b_delta):
    B, S, H = hidden_states.shape
    M = W_mask.shape[1]
    D = W_delta.shape[1]
    num_selected = _NUM_SELECTED_STATIC
    S_BLK = S // 2

    mask_flat, edit_delta = pl.pallas_call(
        functools.partial(
            _edit_head_kernel,
            seq_len=S,
            num_selected_static=num_selected,
        ),
        grid=(B, 2),
        in_specs=[
            pl.BlockSpec((1, S_BLK, H), lambda i, j: (i, j, 0)),
            pl.BlockSpec((H, M), lambda i, j: (0, 0)),
            pl.BlockSpec((M,), lambda i, j: (0,)),
            pl.BlockSpec((H, D), lambda i, j: (0, 0)),
            pl.BlockSpec((D,), lambda i, j: (0,)),
        ],
        out_specs=[
            pl.BlockSpec((1, 1, M), lambda i, j: (i, 0, 0)),
            pl.BlockSpec((1, num_selected, D), lambda i, j: (i, 0, 0)),
        ],
        out_shape=[
            jax.ShapeDtypeStruct((B, 1, M), jnp.float32),
            jax.ShapeDtypeStruct((B, num_selected, D), jnp.float32),
        ],
        scratch_shapes=[pltpu.VMEM((1, H), jnp.float32)],
    )(hidden_states, W_mask, b_mask, W_delta, b_delta)

    mask_size = int(round(M ** 0.5))
    sparse_mask = mask_flat.reshape(B, mask_size, mask_size)
    return sparse_mask, edit_delta


def kernel(hidden_states, W_mask, b_mask, W_delta, b_delta, num_selected):
    del num_selected
    return _edit_head(hidden_states, W_mask, b_mask, W_delta, b_delta)


# speedup vs baseline: 6.4236x; 1.1391x over previous
import functools

import jax
import jax.numpy as jnp
from jax.experimental import pallas as pl

_NUM_SELECTED_STATIC = 256


def _edit_head_kernel(h_ref, wm_ref, bm_ref, wd_ref, bd_ref,
                      mask_out_ref, delta_out_ref,
                      *, seq_len, num_selected_static):
    h = h_ref[0]

    last_hidden = h[-1:, :]
    mask_out_ref[0] = (
        jnp.dot(last_hidden, wm_ref[...],
                preferred_element_type=jnp.float32) + bm_ref[...]
    )

    mean_h = jnp.sum(h, axis=0, keepdims=True) * (1.0 / seq_len)
    delta_row = (
        jnp.dot(mean_h, wd_ref[...],
                preferred_element_type=jnp.float32) + bd_ref[...]
    )
    delta_out_ref[...] = jnp.broadcast_to(
        delta_row[:, None, :], (1, num_selected_static, delta_row.shape[1])
    )


@jax.jit
def _edit_head(hidden_states, W_mask, b_mask, W_delta, b_delta):
    B, S, H = hidden_states.shape
    M = W_mask.shape[1]
    D = W_delta.shape[1]
    num_selected = _NUM_SELECTED_STATIC

    mask_flat, edit_delta = pl.pallas_call(
        functools.partial(
            _edit_head_kernel,
            seq_len=S,
            num_selected_static=num_selected,
        ),
        grid=(B,),
        in_specs=[
            pl.BlockSpec((1, S, H), lambda i: (i, 0, 0)),
            pl.BlockSpec((H, M), lambda i: (0, 0)),
            pl.BlockSpec((M,), lambda i: (0,)),
            pl.BlockSpec((H, D), lambda i: (0, 0)),
            pl.BlockSpec((D,), lambda i: (0,)),
        ],
        out_specs=[
            pl.BlockSpec((1, 1, M), lambda i: (i, 0, 0)),
            pl.BlockSpec((1, num_selected, D), lambda i: (i, 0, 0)),
        ],
        out_shape=[
            jax.ShapeDtypeStruct((B, 1, M), jnp.float32),
            jax.ShapeDtypeStruct((B, num_selected, D), jnp.float32),
        ],
    )(hidden_states, W_mask, b_mask, W_delta, b_delta)

    mask_size = int(round(M ** 0.5))
    sparse_mask = mask_flat.reshape(B, mask_size, mask_size)
    return sparse_mask, edit_delta


def kernel(hidden_states, W_mask, b_mask, W_delta, b_delta, num_selected):
    del num_selected
    return _edit_head(hidden_states, W_mask, b_mask, W_delta, b_delta)
